# baseline (speedup 1.0000x reference)
import jax
import jax.numpy as jnp
from jax import lax
from jax.experimental import pallas as pl
from jax.experimental.pallas import tpu as pltpu
from jax.experimental.pallas import tpu_sc as plsc

N_NODES = 100000
MEM_DIM = 128
MSG_DIM = 128
B = 16384

NC = 2
NS = 16
L = 16
NW = NC * NS

RNG = 3136
LAST = N_NODES - (NW - 1) * RNG
WIN_SIZE = NW * RNG
NCHUNK = B // L
BPW = B // NW

_mesh = plsc.VectorSubcoreMesh(
    core_axis_name="c", subcore_axis_name="s", num_cores=NC, num_subcores=NS
)
_sc_params = pltpu.CompilerParams(needs_layout_passes=False)


def _worker_id():
  return lax.axis_index("s") * NC + lax.axis_index("c")


CPBLK = 10000


def _copy_body(x_ref, o_ref):
  o_ref[...] = x_ref[...]


def _table_copy(table):
  return pl.pallas_call(
      _copy_body,
      grid=(N_NODES // CPBLK,),
      in_specs=[pl.BlockSpec((CPBLK, MSG_DIM), lambda i: (i, 0))],
      out_specs=pl.BlockSpec((CPBLK, MSG_DIM), lambda i: (i, 0)),
      out_shape=jax.ShapeDtypeStruct((N_NODES, MSG_DIM), jnp.float32),
  )(table)


_SENT = 0x40000000


def _dupkill_body(n_ref, o_ref):
  rows = [n_ref[s, :] for s in range(L)]
  for s in range(L - 1):
    kill = rows[s] != rows[s]
    for t in range(s + 1, L):
      kill = kill | (rows[s] == rows[t])
    o_ref[s, :] = jnp.where(kill, _SENT, rows[s])
  o_ref[L - 1, :] = rows[L - 1]


def _dupkill(nodes_lanemajor):
  return pl.pallas_call(
      _dupkill_body,
      out_shape=jax.ShapeDtypeStruct((L, NCHUNK), jnp.int32),
  )(nodes_lanemajor)


def _winner_body(nodes_hbm, ts_in_hbm, win_out, ts_out,
                 nodes_v, win_v, tsbuf, sem):
  wid = _worker_id()
  base = wid * RNG
  pltpu.sync_copy(nodes_hbm, nodes_v)
  lane = lax.iota(jnp.int32, L)

  def chunk(c, carry):
    n = nodes_v[pl.ds(c * L, L)]
    i_vec = c * L + lane
    m = (n >= base) & (n < base + RNG)
    loc = jnp.where(m, n - base, 0)
    plsc.store_scatter(win_v, [loc], i_vec, mask=m)
    return carry

  lax.fori_loop(0, NCHUNK, chunk, 0)
  pltpu.sync_copy(win_v, win_out.at[pl.ds(base, RNG)])

  @pl.when(wid < NW - 1)
  def _():
    pltpu.sync_copy(ts_in_hbm.at[pl.ds(base, RNG)], tsbuf)
    pltpu.sync_copy(tsbuf, ts_out.at[pl.ds(base, RNG)])

  @pl.when(wid == NW - 1)
  def _():
    pltpu.sync_copy(ts_in_hbm.at[pl.ds(base, LAST)], tsbuf.at[pl.ds(0, LAST)])
    pltpu.sync_copy(tsbuf.at[pl.ds(0, LAST)], ts_out.at[pl.ds(base, LAST)])


_winner_kernel = pl.kernel(
    _winner_body,
    out_type=(
        jax.ShapeDtypeStruct((WIN_SIZE,), jnp.int32),
        jax.ShapeDtypeStruct((N_NODES,), jnp.float32),
    ),
    mesh=_mesh,
    compiler_params=_sc_params,
    scratch_types=[
        pltpu.VMEM((B,), jnp.int32),
        pltpu.VMEM((RNG,), jnp.int32),
        pltpu.VMEM((RNG,), jnp.float32),
        pltpu.SemaphoreType.DMA,
    ],
)


BPQ = BPW // 4


def _gout_body(mem_hbm, nodes_hbm, gout_hbm, nodes_v, buf0, buf1, sem0, sem1):
  wid = _worker_id()
  b0 = wid * BPW
  pltpu.sync_copy(nodes_hbm.at[pl.ds(b0, BPW)], nodes_v)
  nq = [nodes_v.at[pl.ds(q * BPQ, BPQ)] for q in range(4)]
  g0 = pltpu.async_copy(mem_hbm.at[nq[0]], buf0, sem0)
  g1 = pltpu.async_copy(mem_hbm.at[nq[1]], buf1, sem1)
  g0.wait()
  w0 = pltpu.async_copy(buf0, gout_hbm.at[pl.ds(b0, BPQ)], sem0)
  g1.wait()
  w1 = pltpu.async_copy(buf1, gout_hbm.at[pl.ds(b0 + BPQ, BPQ)], sem1)
  w0.wait()
  g2 = pltpu.async_copy(mem_hbm.at[nq[2]], buf0, sem0)
  w1.wait()
  g3 = pltpu.async_copy(mem_hbm.at[nq[3]], buf1, sem1)
  g2.wait()
  w2 = pltpu.async_copy(buf0, gout_hbm.at[pl.ds(b0 + 2 * BPQ, BPQ)], sem0)
  g3.wait()
  w3 = pltpu.async_copy(buf1, gout_hbm.at[pl.ds(b0 + 3 * BPQ, BPQ)], sem1)
  w2.wait()
  w3.wait()


_gout_kernel = pl.kernel(
    _gout_body,
    out_type=jax.ShapeDtypeStruct((B, MEM_DIM), jnp.float32),
    mesh=_mesh,
    compiler_params=_sc_params,
    scratch_types=[
        pltpu.VMEM((BPW,), jnp.int32),
        pltpu.VMEM((BPQ, MSG_DIM), jnp.float32),
        pltpu.VMEM((BPQ, MSG_DIM), jnp.float32),
        pltpu.SemaphoreType.DMA,
        pltpu.SemaphoreType.DMA,
    ],
)


def _scatter_body(msg_hbm, ts_hbm, nodes_hbm, win_hbm,
                  table_ref, tsout_ref, dummy_out,
                  nodes_v, win_v, rowsG, tsbuf_v, semB, semC):
  wid = _worker_id()
  b0 = wid * BPW
  pltpu.sync_copy(nodes_hbm.at[pl.ds(b0, BPW)], nodes_v)
  cb1 = pltpu.async_copy(win_hbm.at[nodes_v], win_v, semB)
  cb1.wait()
  cb2 = pltpu.async_copy(msg_hbm.at[win_v], rowsG, semB)
  cc1 = pltpu.async_copy(ts_hbm.at[win_v], tsbuf_v, semC)
  cb2.wait()
  cb3 = pltpu.async_copy(rowsG, table_ref.at[nodes_v], semB)
  cc1.wait()
  cc2 = pltpu.async_copy(tsbuf_v, tsout_ref.at[nodes_v], semC)
  cb3.wait()
  cc2.wait()
  @pl.when(wid == 0)
  def _():
    pltpu.sync_copy(nodes_v.at[pl.ds(0, L)], dummy_out)


_scatter_kernel = pl.kernel(
    _scatter_body,
    out_type=jax.ShapeDtypeStruct((L,), jnp.int32),
    mesh=_mesh,
    compiler_params=_sc_params,
    scratch_types=[
        pltpu.VMEM((BPW,), jnp.int32),
        pltpu.VMEM((BPW,), jnp.int32),
        pltpu.VMEM((BPW, MSG_DIM), jnp.float32),
        pltpu.VMEM((BPW,), jnp.float32),
        pltpu.SemaphoreType.DMA,
        pltpu.SemaphoreType.DMA,
    ],
)


def kernel(memory, messages_table, timestamps_table, messages, timestamps,
           nodes):
  nodes_dk = _dupkill(jnp.reshape(nodes, (NCHUNK, L)).T)
  nodes_dk = jnp.reshape(nodes_dk.T, (B,))
  gathered = _gout_kernel(memory, nodes)
  winner, ts_copy = _winner_kernel(nodes_dk, timestamps_table)
  t_copy = _table_copy(messages_table)
  t_ref = jax.new_ref(t_copy)
  ts_ref = jax.new_ref(ts_copy)
  _ = _scatter_kernel(messages, timestamps, nodes, winner, t_ref, ts_ref)
  return gathered, jax.freeze(t_ref), jax.freeze(ts_ref)


# speedup vs baseline: 1.2221x; 1.2221x over previous
import jax
import jax.numpy as jnp
from jax import lax
from jax.experimental import pallas as pl
from jax.experimental.pallas import tpu as pltpu
from jax.experimental.pallas import tpu_sc as plsc

N_NODES = 100000
MEM_DIM = 128
MSG_DIM = 128
B = 16384

NC = 2
NS = 16
L = 16
NW = NC * NS

RNG = 3136
LAST = N_NODES - (NW - 1) * RNG
WIN_SIZE = NW * RNG
NCHUNK = B // L
BPW = B // NW

_mesh = plsc.VectorSubcoreMesh(
    core_axis_name="c", subcore_axis_name="s", num_cores=NC, num_subcores=NS
)
_sc_params = pltpu.CompilerParams(needs_layout_passes=False)


def _worker_id():
  return lax.axis_index("s") * NC + lax.axis_index("c")


CPBLK = 5000


def _copy_body(x_ref, o_ref):
  o_ref[...] = x_ref[...]


def _table_copy(table):
  return pl.pallas_call(
      _copy_body,
      grid=(N_NODES // CPBLK,),
      in_specs=[pl.BlockSpec((CPBLK, MSG_DIM), lambda i: (i, 0))],
      out_specs=pl.BlockSpec((CPBLK, MSG_DIM), lambda i: (i, 0)),
      out_shape=jax.ShapeDtypeStruct((N_NODES, MSG_DIM), jnp.float32),
  )(table)


_SENT = 0x40000000


def _dupkill_body(n_ref, o_ref):
  rows = [n_ref[s, :] for s in range(L)]
  for s in range(L - 1):
    kill = rows[s] != rows[s]
    for t in range(s + 1, L):
      kill = kill | (rows[s] == rows[t])
    o_ref[s, :] = jnp.where(kill, _SENT, rows[s])
  o_ref[L - 1, :] = rows[L - 1]


def _dupkill(nodes_lanemajor):
  return pl.pallas_call(
      _dupkill_body,
      out_shape=jax.ShapeDtypeStruct((L, NCHUNK), jnp.int32),
  )(nodes_lanemajor)


def _winner_body(nodes_hbm, ts_in_hbm, tsb_hbm, win_out, ts_out,
                 nodes_v, win_v, tsb_v, tsloc, sem):
  wid = _worker_id()
  base = wid * RNG
  pltpu.sync_copy(nodes_hbm, nodes_v)
  pltpu.sync_copy(tsb_hbm, tsb_v)
  @pl.when(wid < NW - 1)
  def _():
    pltpu.sync_copy(ts_in_hbm.at[pl.ds(base, RNG)], tsloc)

  @pl.when(wid == NW - 1)
  def _():
    pltpu.sync_copy(ts_in_hbm.at[pl.ds(base, LAST)], tsloc.at[pl.ds(0, LAST)])

  lane = lax.iota(jnp.int32, L)

  def chunk(c, carry):
    n = nodes_v[pl.ds(c * L, L)]
    i_vec = c * L + lane
    m = (n >= base) & (n < base + RNG)
    loc = jnp.where(m, n - base, 0)
    plsc.store_scatter(win_v, [loc], i_vec, mask=m)
    tsv = tsb_v[pl.ds(c * L, L)]
    plsc.store_scatter(tsloc, [loc], tsv, mask=m)
    return carry

  lax.fori_loop(0, NCHUNK, chunk, 0)
  pltpu.sync_copy(win_v, win_out.at[pl.ds(base, RNG)])

  @pl.when(wid < NW - 1)
  def _():
    pltpu.sync_copy(tsloc, ts_out.at[pl.ds(base, RNG)])

  @pl.when(wid == NW - 1)
  def _():
    pltpu.sync_copy(tsloc.at[pl.ds(0, LAST)], ts_out.at[pl.ds(base, LAST)])


_winner_kernel = pl.kernel(
    _winner_body,
    out_type=(
        jax.ShapeDtypeStruct((WIN_SIZE,), jnp.int32),
        jax.ShapeDtypeStruct((N_NODES,), jnp.float32),
    ),
    mesh=_mesh,
    compiler_params=_sc_params,
    scratch_types=[
        pltpu.VMEM((B,), jnp.int32),
        pltpu.VMEM((RNG,), jnp.int32),
        pltpu.VMEM((B,), jnp.float32),
        pltpu.VMEM((RNG,), jnp.float32),
        pltpu.SemaphoreType.DMA,
    ],
)


BPQ = BPW // 4


def _gout_body(mem_hbm, nodes_hbm, gout_hbm, nodes_v, buf0, buf1, sem0, sem1):
  wid = _worker_id()
  b0 = wid * BPW
  pltpu.sync_copy(nodes_hbm.at[pl.ds(b0, BPW)], nodes_v)
  nq = [nodes_v.at[pl.ds(q * BPQ, BPQ)] for q in range(4)]
  g0 = pltpu.async_copy(mem_hbm.at[nq[0]], buf0, sem0)
  g1 = pltpu.async_copy(mem_hbm.at[nq[1]], buf1, sem1)
  g0.wait()
  w0 = pltpu.async_copy(buf0, gout_hbm.at[pl.ds(b0, BPQ)], sem0)
  g1.wait()
  w1 = pltpu.async_copy(buf1, gout_hbm.at[pl.ds(b0 + BPQ, BPQ)], sem1)
  w0.wait()
  g2 = pltpu.async_copy(mem_hbm.at[nq[2]], buf0, sem0)
  w1.wait()
  g3 = pltpu.async_copy(mem_hbm.at[nq[3]], buf1, sem1)
  g2.wait()
  w2 = pltpu.async_copy(buf0, gout_hbm.at[pl.ds(b0 + 2 * BPQ, BPQ)], sem0)
  g3.wait()
  w3 = pltpu.async_copy(buf1, gout_hbm.at[pl.ds(b0 + 3 * BPQ, BPQ)], sem1)
  w2.wait()
  w3.wait()


_gout_kernel = pl.kernel(
    _gout_body,
    out_type=jax.ShapeDtypeStruct((B, MEM_DIM), jnp.float32),
    mesh=_mesh,
    compiler_params=_sc_params,
    scratch_types=[
        pltpu.VMEM((BPW,), jnp.int32),
        pltpu.VMEM((BPQ, MSG_DIM), jnp.float32),
        pltpu.VMEM((BPQ, MSG_DIM), jnp.float32),
        pltpu.SemaphoreType.DMA,
        pltpu.SemaphoreType.DMA,
    ],
)


def _scatter_body(msg_hbm, nodes_hbm, win_hbm,
                  table_ref, dummy_out,
                  nodes_v, win_v, rowsG, semB):
  wid = _worker_id()
  b0 = wid * BPW
  pltpu.sync_copy(nodes_hbm.at[pl.ds(b0, BPW)], nodes_v)
  cb1 = pltpu.async_copy(win_hbm.at[nodes_v], win_v, semB)
  cb1.wait()
  cb2 = pltpu.async_copy(msg_hbm.at[win_v], rowsG, semB)
  cb2.wait()
  cb3 = pltpu.async_copy(rowsG, table_ref.at[nodes_v], semB)
  cb3.wait()
  @pl.when(wid == 0)
  def _():
    pltpu.sync_copy(nodes_v.at[pl.ds(0, L)], dummy_out)


_scatter_kernel = pl.kernel(
    _scatter_body,
    out_type=jax.ShapeDtypeStruct((L,), jnp.int32),
    mesh=_mesh,
    compiler_params=_sc_params,
    scratch_types=[
        pltpu.VMEM((BPW,), jnp.int32),
        pltpu.VMEM((BPW,), jnp.int32),
        pltpu.VMEM((BPW, MSG_DIM), jnp.float32),
        pltpu.SemaphoreType.DMA,
    ],
)


def kernel(memory, messages_table, timestamps_table, messages, timestamps,
           nodes):
  nodes_dk = _dupkill(jnp.reshape(nodes, (NCHUNK, L)).T)
  nodes_dk = jnp.reshape(nodes_dk.T, (B,))
  gathered = _gout_kernel(memory, nodes)
  winner, ts_new = _winner_kernel(nodes_dk, timestamps_table, timestamps)
  t_copy = _table_copy(messages_table)
  t_ref = jax.new_ref(t_copy)
  _ = _scatter_kernel(messages, nodes, winner, t_ref)
  return gathered, jax.freeze(t_ref), ts_new
